# baseline (speedup 1.0000x reference)
import jax
import jax.numpy as jnp
from jax import lax
from jax.experimental import pallas as pl
from jax.experimental.pallas import tpu as pltpu
from jax.experimental.pallas import tpu_sc as plsc

VOCAB = 100000
EMBED = 64
CPW = 50
L = 16
CHUNKS = VOCAB * CPW // L
WIN = 6

_info = plsc.get_sparse_core_info()
_NC = _info.num_cores


def _body(entity_hbm, w2c16_hbm, wemb_hbm, cemb_hbm,
          out_l, out_r, out_cl, out_cr,
          ent_v, win_v, candidx_v, wrows_v, cl_v, cr_v, sem, sem2):
    wid = lax.axis_index("s") * _NC + lax.axis_index("c")

    @pl.when(wid == 0)
    def _():
        pltpu.sync_copy(entity_hbm, ent_v.at[pl.ds(0, 4)])

        word_cp = pltpu.async_copy(
            wemb_hbm.at[ent_v.at[pl.ds(0, 4)]], wrows_v, sem2)

        iota = lax.iota(jnp.int32, L)
        ent_vec = ent_v[...]
        e0 = jnp.sum(jnp.where(iota == 0, ent_vec, 0))
        e1 = jnp.sum(jnp.where(iota == 1, ent_vec, 0))

        flat_l = e0 * CPW
        flat_r = e1 * CPW
        start_l = jnp.minimum(flat_l >> 4, CHUNKS - WIN)
        start_r = jnp.minimum(flat_r >> 4, CHUNKS - WIN)
        off_l = flat_l - start_l * L
        off_r = flat_r - start_r * L

        cp_l = pltpu.async_copy(
            w2c16_hbm.at[pl.ds(start_l, WIN)], win_v.at[pl.ds(0, WIN)], sem)
        cp_r = pltpu.async_copy(
            w2c16_hbm.at[pl.ds(start_r, WIN)], win_v.at[pl.ds(WIN, WIN)], sem)
        cp_l.wait()
        cp_r.wait()

        for s, off in ((0, off_l), (1, off_r)):
            base = jnp.full((L,), s * WIN * L, jnp.int32)
            for k in range(4):
                f = jnp.minimum(off + iota + k * L, WIN * L - 1) + base
                candidx_v[s, pl.ds(k * L, L)] = plsc.load_gather(
                    win_v, [f >> 4, f & (L - 1)])

        cp_cl = pltpu.async_copy(cemb_hbm.at[candidx_v.at[0]], cl_v, sem)
        cp_cr = pltpu.async_copy(cemb_hbm.at[candidx_v.at[1]], cr_v, sem)
        cp_cl.wait()
        cp_cr.wait()
        word_cp.wait()

        pltpu.sync_copy(wrows_v.at[pl.ds(2, 1)], out_l)
        pltpu.sync_copy(wrows_v.at[pl.ds(3, 1)], out_r)
        pltpu.sync_copy(cl_v.at[pl.ds(0, CPW)], out_cl)
        pltpu.sync_copy(cr_v.at[pl.ds(0, CPW)], out_cr)


_sc_call = pl.kernel(
    _body,
    out_type=(
        jax.ShapeDtypeStruct((1, EMBED), jnp.float32),
        jax.ShapeDtypeStruct((1, EMBED), jnp.float32),
        jax.ShapeDtypeStruct((CPW, EMBED), jnp.float32),
        jax.ShapeDtypeStruct((CPW, EMBED), jnp.float32),
    ),
    mesh=plsc.VectorSubcoreMesh(core_axis_name="c", subcore_axis_name="s"),
    compiler_params=pltpu.CompilerParams(use_tc_tiling_on_sc=False,
                                         needs_layout_passes=False),
    scratch_types=[
        pltpu.VMEM((L,), jnp.int32),
        pltpu.VMEM((2 * WIN, L), jnp.int32),
        pltpu.VMEM((2, 4 * L), jnp.int32),
        pltpu.VMEM((4, EMBED), jnp.float32),
        pltpu.VMEM((4 * L, EMBED), jnp.float32),
        pltpu.VMEM((4 * L, EMBED), jnp.float32),
        pltpu.SemaphoreType.DMA,
        pltpu.SemaphoreType.DMA,
    ],
)


def kernel(entity, word2candidates, word_embeds, candidate_embeds):
    w2c16 = word2candidates.reshape(CHUNKS, L)
    return _sc_call(entity, w2c16, word_embeds, candidate_embeds)


# speedup vs baseline: 6.3795x; 6.3795x over previous
import jax
import jax.numpy as jnp
from jax import lax
from jax.experimental import pallas as pl
from jax.experimental.pallas import tpu as pltpu
from jax.experimental.pallas import tpu_sc as plsc

VOCAB = 100000
EMBED = 64
CPW = 50
L = 16

_info = plsc.get_sparse_core_info()
_NC = _info.num_cores


def _extract_col(emb_v, lane_vec, out_row_ref):
    iota = lax.iota(jnp.int32, L)
    for k in range(4):
        out_row_ref[pl.ds(k * L, L)] = plsc.load_gather(
            emb_v, [iota + k * L, lane_vec])


def _body(ent_hbm, w2c_t, wemb_t, cemb_t,
          wemb_out, cand_out,
          ent_v, widx_v, emb_v, blk_v, sem):
    wid = lax.axis_index("s") * _NC + lax.axis_index("c")
    side = wid & 1
    tp = wid >> 1

    pltpu.sync_copy(ent_hbm, ent_v.at[pl.ds(0, 4)])
    iota = lax.iota(jnp.int32, L)
    ent_vec = ent_v[...]
    zeros = jnp.zeros((L,), jnp.int32)

    @pl.when(tp <= 6)
    def _cand():
        e = jnp.sum(jnp.where(iota == side, ent_vec, 0))
        c0w = pl.multiple_of((e >> 7) << 7, 128)
        lw = zeros + (e & 127)
        row_start = pl.multiple_of(8 * tp, 8)
        pltpu.sync_copy(
            w2c_t.at[pl.ds(row_start, 8), pl.ds(c0w, 128)], widx_v)
        for i in range(8):
            idv = plsc.load_gather(widx_v.at[i], [lw])
            rid = jnp.sum(jnp.where(iota == 0, idv, 0))
            rid = jnp.clip(rid, 0, VOCAB - 1)
            c0 = pl.multiple_of((rid >> 7) << 7, 128)
            lc = zeros + (rid & 127)
            pltpu.sync_copy(cemb_t.at[:, pl.ds(c0, 128)], emb_v)
            _extract_col(emb_v, lc, blk_v.at[i])
        out_start = pl.multiple_of(side * 64 + 8 * tp, 8)
        pltpu.sync_copy(blk_v, cand_out.at[pl.ds(out_start, 8), :])

    @pl.when(tp == 15)
    def _word():
        e = jnp.sum(jnp.where(iota == 2 + side, ent_vec, 0))
        c0 = pl.multiple_of((e >> 7) << 7, 128)
        lw = zeros + (e & 127)
        pltpu.sync_copy(wemb_t.at[:, pl.ds(c0, 128)], emb_v)
        _extract_col(emb_v, lw, blk_v.at[0])
        out_start = pl.multiple_of(side * 8, 8)
        pltpu.sync_copy(blk_v, wemb_out.at[pl.ds(out_start, 8), :])


_sc_call = pl.kernel(
    _body,
    out_type=(
        jax.ShapeDtypeStruct((16, 128), jnp.float32),
        jax.ShapeDtypeStruct((128, 128), jnp.float32),
    ),
    mesh=plsc.VectorSubcoreMesh(core_axis_name="c", subcore_axis_name="s"),
    compiler_params=pltpu.CompilerParams(use_tc_tiling_on_sc=True,
                                         needs_layout_passes=False),
    scratch_types=[
        pltpu.VMEM((L,), jnp.int32),
        pltpu.VMEM((8, 128), jnp.int32),
        pltpu.VMEM((64, 128), jnp.float32),
        pltpu.VMEM((8, 128), jnp.float32),
        pltpu.SemaphoreType.DMA,
    ],
)


def kernel(entity, word2candidates, word_embeds, candidate_embeds):
    wemb_wide, cand_wide = _sc_call(
        entity, word2candidates.T, word_embeds.T, candidate_embeds.T)
    out_l = wemb_wide[0:1, :EMBED]
    out_r = wemb_wide[8:9, :EMBED]
    out_cl = cand_wide[:CPW, :EMBED]
    out_cr = cand_wide[64:64 + CPW, :EMBED]
    return (out_l, out_r, out_cl, out_cr)


# speedup vs baseline: 7.5183x; 1.1785x over previous
import jax
import jax.numpy as jnp
from jax import lax
from jax.experimental import pallas as pl
from jax.experimental.pallas import tpu as pltpu
from jax.experimental.pallas import tpu_sc as plsc

VOCAB = 100000
EMBED = 64
CPW = 50
L = 16

_info = plsc.get_sparse_core_info()
_NC = _info.num_cores


def _extract_col(emb_ref, lane_vec, out_row_ref):
    iota = lax.iota(jnp.int32, L)
    for k in range(4):
        out_row_ref[pl.ds(k * L, L)] = plsc.load_gather(
            emb_ref, [iota + k * L, lane_vec])


def _body(ent_hbm, w2c_t, wemb_t, cemb_t,
          wemb_out, cand_out,
          ent_v, widx_v, emb_v, blk_v, sem, sem2):
    wid = lax.axis_index("s") * _NC + lax.axis_index("c")
    side = wid & 1
    tp = wid >> 1

    pltpu.sync_copy(ent_hbm, ent_v.at[pl.ds(0, 4)])
    iota = lax.iota(jnp.int32, L)
    ent_vec = ent_v[...]
    zeros = jnp.zeros((L,), jnp.int32)

    @pl.when(tp <= 6)
    def _cand():
        e = jnp.sum(jnp.where(iota == side, ent_vec, 0))
        c0w = pl.multiple_of((e >> 7) << 7, 128)
        lw = zeros + (e & 127)
        row_start = pl.multiple_of(8 * tp, 8)
        pltpu.sync_copy(
            w2c_t.at[pl.ds(row_start, 8), pl.ds(c0w, 128)], widx_v)
        lanes = []
        copies = []
        for i in range(8):
            idv = plsc.load_gather(widx_v.at[i], [lw])
            rid = jnp.sum(jnp.where(iota == 0, idv, 0))
            rid = jnp.clip(rid, 0, VOCAB - 1)
            c0 = pl.multiple_of((rid >> 7) << 7, 128)
            lanes.append(zeros + (rid & 127))
            copies.append(pltpu.async_copy(
                cemb_t.at[:, pl.ds(c0, 128)], emb_v.at[i], sem))
        for i in range(8):
            copies[i].wait()
            _extract_col(emb_v.at[i], lanes[i], blk_v.at[i])
        out_start = pl.multiple_of(side * 64 + 8 * tp, 8)
        pltpu.sync_copy(blk_v, cand_out.at[pl.ds(out_start, 8), :])

    @pl.when(tp == 15)
    def _word():
        e = jnp.sum(jnp.where(iota == 2 + side, ent_vec, 0))
        c0 = pl.multiple_of((e >> 7) << 7, 128)
        lw = zeros + (e & 127)
        pltpu.async_copy(wemb_t.at[:, pl.ds(c0, 128)], emb_v.at[0],
                         sem2).wait()
        _extract_col(emb_v.at[0], lw, blk_v.at[0])
        out_start = pl.multiple_of(side * 8, 8)
        pltpu.sync_copy(blk_v, wemb_out.at[pl.ds(out_start, 8), :])


_sc_call = pl.kernel(
    _body,
    out_type=(
        jax.ShapeDtypeStruct((16, 128), jnp.float32),
        jax.ShapeDtypeStruct((128, 128), jnp.float32),
    ),
    mesh=plsc.VectorSubcoreMesh(core_axis_name="c", subcore_axis_name="s"),
    compiler_params=pltpu.CompilerParams(use_tc_tiling_on_sc=True,
                                         needs_layout_passes=False),
    scratch_types=[
        pltpu.VMEM((L,), jnp.int32),
        pltpu.VMEM((8, 128), jnp.int32),
        pltpu.VMEM((8, 64, 128), jnp.float32),
        pltpu.VMEM((8, 128), jnp.float32),
        pltpu.SemaphoreType.DMA,
        pltpu.SemaphoreType.DMA,
    ],
)


def kernel(entity, word2candidates, word_embeds, candidate_embeds):
    wemb_wide, cand_wide = _sc_call(
        entity, word2candidates.T, word_embeds.T, candidate_embeds.T)
    out_l = wemb_wide[0:1, :EMBED]
    out_r = wemb_wide[8:9, :EMBED]
    out_cl = cand_wide[:CPW, :EMBED]
    out_cr = cand_wide[64:64 + CPW, :EMBED]
    return (out_l, out_r, out_cl, out_cr)


# speedup vs baseline: 8.6197x; 1.1465x over previous
import jax
import jax.numpy as jnp
from jax import lax
from jax.experimental import pallas as pl
from jax.experimental.pallas import tpu as pltpu
from jax.experimental.pallas import tpu_sc as plsc

VOCAB = 100000
EMBED = 64
CPW = 50
L = 16

_info = plsc.get_sparse_core_info()
_NC = _info.num_cores


def _extract_col64(emb_ref, lane_vec, out_row_ref):
    iota = lax.iota(jnp.int32, L)
    for k in range(4):
        out_row_ref[pl.ds(k * L, L)] = plsc.load_gather(
            emb_ref, [iota + k * L, lane_vec])


def _body(ent_hbm, w2c_t, wemb_t, cemb_t,
          out_l, out_r, out_cl, out_cr,
          ent_v, widx_v, emb_v, blk_v, blk64_v, sem, sem2):
    wid = lax.axis_index("s") * _NC + lax.axis_index("c")
    side = wid & 1
    tp = wid >> 1

    pltpu.sync_copy(ent_hbm, ent_v.at[pl.ds(0, 4)])
    iota = lax.iota(jnp.int32, L)
    ent_vec = ent_v[...]
    zeros = jnp.zeros((L,), jnp.int32)

    @pl.when(tp <= 6)
    def _cand():
        e = jnp.sum(jnp.where(iota == side, ent_vec, 0))
        c0w = pl.multiple_of((e >> 7) << 7, 128)
        lw = zeros + (e & 127)
        row_start = pl.multiple_of(8 * tp, 8)
        pltpu.sync_copy(
            w2c_t.at[pl.ds(row_start, 8), pl.ds(c0w, 128)], widx_v)
        lanes = []
        copies = []
        for i in range(8):
            idv = plsc.load_gather(widx_v.at[i], [lw])
            rid = jnp.sum(jnp.where(iota == 0, idv, 0))
            rid = jnp.clip(rid, 0, VOCAB - 1)
            c0 = pl.multiple_of((rid >> 7) << 7, 128)
            lanes.append(zeros + (rid & 127))
            copies.append(pltpu.async_copy(
                cemb_t.at[:, pl.ds(c0, 128)], emb_v.at[i], sem))
        for i in range(8):
            copies[i].wait()
            _extract_col64(emb_v.at[i], lanes[i], blk64_v.at[i])
        out_start = pl.multiple_of(8 * tp, 8)

        @pl.when(side == 0)
        def _():
            pltpu.sync_copy(blk64_v, out_cl.at[pl.ds(out_start, 8), :])

        @pl.when(side == 1)
        def _():
            pltpu.sync_copy(blk64_v, out_cr.at[pl.ds(out_start, 8), :])

    @pl.when(tp == 15)
    def _word():
        e = jnp.sum(jnp.where(iota == 2 + side, ent_vec, 0))
        c0 = pl.multiple_of((e >> 7) << 7, 128)
        lw = zeros + (e & 127)
        pltpu.async_copy(wemb_t.at[:, pl.ds(c0, 128)], emb_v.at[0],
                         sem2).wait()
        _extract_col64(emb_v.at[0], lw, blk64_v.at[0])

        @pl.when(side == 0)
        def _():
            pltpu.sync_copy(blk64_v.at[pl.ds(0, 1), :], out_l)

        @pl.when(side == 1)
        def _():
            pltpu.sync_copy(blk64_v.at[pl.ds(0, 1), :], out_r)


_sc_call = pl.kernel(
    _body,
    out_type=(
        jax.ShapeDtypeStruct((1, EMBED), jnp.float32),
        jax.ShapeDtypeStruct((1, EMBED), jnp.float32),
        jax.ShapeDtypeStruct((CPW, EMBED), jnp.float32),
        jax.ShapeDtypeStruct((CPW, EMBED), jnp.float32),
    ),
    mesh=plsc.VectorSubcoreMesh(core_axis_name="c", subcore_axis_name="s"),
    compiler_params=pltpu.CompilerParams(use_tc_tiling_on_sc=True,
                                         needs_layout_passes=False),
    scratch_types=[
        pltpu.VMEM((L,), jnp.int32),
        pltpu.VMEM((8, 128), jnp.int32),
        pltpu.VMEM((8, 64, 128), jnp.float32),
        pltpu.VMEM((8, 128), jnp.float32),
        pltpu.VMEM((8, EMBED), jnp.float32),
        pltpu.SemaphoreType.DMA,
        pltpu.SemaphoreType.DMA,
    ],
)


def kernel(entity, word2candidates, word_embeds, candidate_embeds):
    return _sc_call(
        entity, word2candidates.T, word_embeds.T, candidate_embeds.T)
